# baseline (speedup 1.0000x reference)
import functools

import jax
import jax.numpy as jnp
from jax.experimental import pallas as pl
from jax.experimental.pallas import tpu as pltpu

SCALE = 28.0
ALPHA = 0.6
BETA = 0.4
SCALE_POS = 10.0
SCALE_NEG = 40.0
LOG2E = 1.4426950408889634
LN2 = 0.6931471805599453


def _loss_kernel(num_classes, grid_n, tile_c,
                 x_ref, w_ref, lab_ref, labrow_ref, tt_ref,
                 out_ref, acc_se_ref, acc_ll_ref, dots_ref, ws_ref):
    i = pl.program_id(0)
    cur = jax.lax.rem(i, 2)

    @pl.when(i == 0)
    def _init():
        acc_se_ref[...] = jnp.zeros_like(acc_se_ref)
        acc_ll_ref[...] = jnp.zeros_like(acc_ll_ref)
        dots_ref[0] = jnp.zeros_like(dots_ref[0])
        ws_ref[1] = jnp.zeros_like(ws_ref[1])

    ws_prev = ws_ref[1 - cur]

    prev = dots_ref[cur]
    base = (i - 2) * tile_c
    iota128 = jax.lax.broadcasted_iota(jnp.int32, (1, 128), 1)
    se_r = None
    ll_r = None
    for k in range(tile_c // 128):
        s = prev[:, k * 128:(k + 1) * 128]
        m = lab_ref[...] == (iota128 + (base + k * 128))
        e_k = jnp.exp2(s)
        l_k = jnp.where(m, s, 0.0)
        se_r = e_k if se_r is None else se_r + e_k
        ll_r = l_k if ll_r is None else ll_r + l_k
    acc_se_ref[...] += se_r
    acc_ll_ref[...] += ll_r

    dots_ref[1 - cur] = jax.lax.dot_general(
        x_ref[...], ws_prev,
        (((1,), (0,)), ((), ())),
        preferred_element_type=jnp.float32)

    cid = jax.lax.broadcasted_iota(jnp.int32, (1, tile_c), 1) + i * tile_c
    valid = cid < num_classes
    w = jnp.where(valid, w_ref[...], 0.0)
    sumsq = jnp.sum(w * w, axis=0, keepdims=True)
    scale = (SCALE * LOG2E) * jax.lax.rsqrt(jnp.maximum(sumsq, 1e-30))
    ws_ref[cur] = (w * scale).astype(jnp.bfloat16)

    @pl.when(i == grid_n + 1)
    def _finalize():
        b = x_ref.shape[0] // 2
        n_extra = grid_n * tile_c - num_classes + 2 * tile_c
        se = jnp.sum(acc_se_ref[...], axis=1, keepdims=True) - n_extra
        ll = jnp.sum(acc_ll_ref[...], axis=1, keepdims=True)
        ce = jnp.log(se) - ll * LN2
        v_loss = jnp.sum(ce[:b]) / b
        t_loss = jnp.sum(ce[b:]) / b

        sim = jax.lax.dot_general(
            x_ref[0:b, :], tt_ref[...],
            (((1,), (0,)), ((), ())),
            preferred_element_type=jnp.float32)
        lmat = lab_ref[0:b, :] == labrow_ref[0:1, :]
        loss_pos = jnp.log1p(jnp.exp(-SCALE_POS * (sim - ALPHA)))
        loss_neg = jnp.log1p(jnp.exp(SCALE_NEG * (sim - BETA)))
        ga = 2.0 * jnp.sum(jnp.where(lmat, loss_pos, loss_neg)) / b

        col = jax.lax.broadcasted_iota(jnp.int32, (8, 128), 1)
        res = jnp.where(col == 0, v_loss + t_loss,
              jnp.where(col == 1, ga,
              jnp.where(col == 2, v_loss, t_loss)))
        out_ref[...] = res


def kernel(visual_embed, textual_embed, labels, W):
    batch, feat = visual_embed.shape
    num_classes = W.shape[1]
    tile_c = 1024
    grid_n = (num_classes + tile_c - 1) // tile_c

    x = jnp.concatenate([visual_embed, textual_embed], axis=0)
    x = x.astype(jnp.bfloat16)
    tt = textual_embed.T.astype(jnp.bfloat16)
    lab = jnp.concatenate([labels, labels]).astype(jnp.int32)
    lab = lab.reshape(2 * batch, 1)
    labrow = jnp.broadcast_to(labels.astype(jnp.int32)[None, :], (8, batch))

    out = pl.pallas_call(
        functools.partial(_loss_kernel, num_classes, grid_n, tile_c),
        grid=(grid_n + 2,),
        in_specs=[
            pl.BlockSpec((2 * batch, feat), lambda i: (0, 0)),
            pl.BlockSpec((feat, tile_c),
                         lambda i: (0, jnp.minimum(i, grid_n - 1))),
            pl.BlockSpec((2 * batch, 1), lambda i: (0, 0)),
            pl.BlockSpec((8, batch), lambda i: (0, 0)),
            pl.BlockSpec((feat, batch), lambda i: (0, 0)),
        ],
        out_specs=pl.BlockSpec((8, 128), lambda i: (0, 0)),
        out_shape=jax.ShapeDtypeStruct((8, 128), jnp.float32),
        scratch_shapes=[
            pltpu.VMEM((2 * batch, 128), jnp.float32),
            pltpu.VMEM((2 * batch, 128), jnp.float32),
            pltpu.VMEM((2, 2 * batch, tile_c), jnp.float32),
            pltpu.VMEM((2, feat, tile_c), jnp.bfloat16),
        ],
    )(x, W, lab, labrow, tt)

    instance_loss = out[0, 0]
    global_align_loss = out[0, 1]
    v_loss = out[0, 2]
    t_loss = out[0, 3]
    return (instance_loss, global_align_loss, v_loss, t_loss)


# speedup vs baseline: 1.1497x; 1.1497x over previous
import functools

import jax
import jax.numpy as jnp
from jax.experimental import pallas as pl
from jax.experimental.pallas import tpu as pltpu

SCALE = 28.0
ALPHA = 0.6
BETA = 0.4
SCALE_POS = 10.0
SCALE_NEG = 40.0
LOG2E = 1.4426950408889634
LN2 = 0.6931471805599453


def _loss_kernel(num_classes, grid_n, tile_c,
                 x_ref, w_ref, lab_ref, labrow_ref, tt_ref,
                 out_ref, acc_se_ref, acc_ll_ref, dots_ref):
    i = pl.program_id(0)
    cur = jax.lax.rem(i, 2)

    @pl.when(i == 0)
    def _init():
        acc_se_ref[...] = jnp.zeros_like(acc_se_ref)
        acc_ll_ref[...] = jnp.zeros_like(acc_ll_ref)
        dots_ref[1] = jnp.zeros_like(dots_ref[1])

    cid = jax.lax.broadcasted_iota(jnp.int32, (1, tile_c), 1) + i * tile_c
    valid = cid < num_classes
    w = jnp.where(valid, w_ref[...], 0.0)
    sumsq = jnp.sum(w * w, axis=0, keepdims=True)
    scale = (SCALE * LOG2E) * jax.lax.rsqrt(jnp.maximum(sumsq, 1e-30))
    ws = (w * scale).astype(jnp.bfloat16)
    dots_ref[cur] = jax.lax.dot_general(
        x_ref[...], ws,
        (((1,), (0,)), ((), ())),
        preferred_element_type=jnp.float32)

    prev = dots_ref[1 - cur]
    base = (i - 1) * tile_c
    iota128 = jax.lax.broadcasted_iota(jnp.int32, (1, 128), 1)
    se_r = None
    ll_r = None
    for k in range(tile_c // 128):
        s = prev[:, k * 128:(k + 1) * 128]
        m = lab_ref[...] == (iota128 + (base + k * 128))
        e_k = jnp.exp2(s)
        l_k = jnp.where(m, s, 0.0)
        se_r = e_k if se_r is None else se_r + e_k
        ll_r = l_k if ll_r is None else ll_r + l_k
    acc_se_ref[...] += se_r
    acc_ll_ref[...] += ll_r

    @pl.when(i == grid_n)
    def _finalize():
        b = x_ref.shape[0] // 2
        n_extra = grid_n * tile_c - num_classes + tile_c
        se = jnp.sum(acc_se_ref[...], axis=1, keepdims=True) - n_extra
        ll = jnp.sum(acc_ll_ref[...], axis=1, keepdims=True)
        ce = jnp.log(se) - ll * LN2
        v_loss = jnp.sum(ce[:b]) / b
        t_loss = jnp.sum(ce[b:]) / b

        sim = jax.lax.dot_general(
            x_ref[0:b, :], tt_ref[...],
            (((1,), (0,)), ((), ())),
            preferred_element_type=jnp.float32)
        lmat = lab_ref[0:b, :] == labrow_ref[0:1, :]
        loss_pos = jnp.log1p(jnp.exp(-SCALE_POS * (sim - ALPHA)))
        loss_neg = jnp.log1p(jnp.exp(SCALE_NEG * (sim - BETA)))
        ga = 2.0 * jnp.sum(jnp.where(lmat, loss_pos, loss_neg)) / b

        col = jax.lax.broadcasted_iota(jnp.int32, (8, 128), 1)
        res = jnp.where(col == 0, v_loss + t_loss,
              jnp.where(col == 1, ga,
              jnp.where(col == 2, v_loss, t_loss)))
        out_ref[...] = res


def kernel(visual_embed, textual_embed, labels, W):
    batch, feat = visual_embed.shape
    num_classes = W.shape[1]
    tile_c = 2048
    grid_n = (num_classes + tile_c - 1) // tile_c

    x = jnp.concatenate([visual_embed, textual_embed], axis=0)
    x = x.astype(jnp.bfloat16)
    tt = textual_embed.T.astype(jnp.bfloat16)
    lab = jnp.concatenate([labels, labels]).astype(jnp.int32)
    lab = lab.reshape(2 * batch, 1)
    labrow = jnp.broadcast_to(labels.astype(jnp.int32)[None, :], (8, batch))

    out = pl.pallas_call(
        functools.partial(_loss_kernel, num_classes, grid_n, tile_c),
        grid=(grid_n + 1,),
        in_specs=[
            pl.BlockSpec((2 * batch, feat), lambda i: (0, 0)),
            pl.BlockSpec((feat, tile_c),
                         lambda i: (0, jnp.minimum(i, grid_n - 1))),
            pl.BlockSpec((2 * batch, 1), lambda i: (0, 0)),
            pl.BlockSpec((8, batch), lambda i: (0, 0)),
            pl.BlockSpec((feat, batch), lambda i: (0, 0)),
        ],
        out_specs=pl.BlockSpec((8, 128), lambda i: (0, 0)),
        out_shape=jax.ShapeDtypeStruct((8, 128), jnp.float32),
        scratch_shapes=[
            pltpu.VMEM((2 * batch, 128), jnp.float32),
            pltpu.VMEM((2 * batch, 128), jnp.float32),
            pltpu.VMEM((2, 2 * batch, tile_c), jnp.float32),
        ],
    )(x, W, lab, labrow, tt)

    instance_loss = out[0, 0]
    global_align_loss = out[0, 1]
    v_loss = out[0, 2]
    t_loss = out[0, 3]
    return (instance_loss, global_align_loss, v_loss, t_loss)


# speedup vs baseline: 1.2471x; 1.0847x over previous
import functools

import jax
import jax.numpy as jnp
from jax.experimental import pallas as pl
from jax.experimental.pallas import tpu as pltpu

SCALE = 28.0
ALPHA = 0.6
BETA = 0.4
SCALE_POS = 10.0
SCALE_NEG = 40.0
LOG2E = 1.4426950408889634
LN2 = 0.6931471805599453


def _loss_kernel(num_classes, grid_n, tile_c,
                 x_ref, w_ref, lab_ref, labrow_ref, tt_ref,
                 out_ref, acc_se_ref, acc_ll_ref, dots_ref):
    i = pl.program_id(0)
    cur = jax.lax.rem(i, 2)

    @pl.when(i == 0)
    def _init():
        acc_se_ref[...] = jnp.zeros_like(acc_se_ref)
        acc_ll_ref[...] = jnp.zeros_like(acc_ll_ref)
        dots_ref[1] = jnp.zeros_like(dots_ref[1])

    cid = jax.lax.broadcasted_iota(jnp.int32, (1, tile_c), 1) + i * tile_c
    valid = cid < num_classes
    w = jnp.where(valid, w_ref[...], 0.0)
    sumsq = jnp.sum(w * w, axis=0, keepdims=True)
    scale = (SCALE * LOG2E) * jax.lax.rsqrt(jnp.maximum(sumsq, 1e-30))
    ws = (w * scale).astype(jnp.bfloat16)
    dots_ref[cur] = jax.lax.dot_general(
        x_ref[...], ws,
        (((1,), (0,)), ((), ())),
        preferred_element_type=jnp.float32)

    b = x_ref.shape[0] // 2
    prev = dots_ref[1 - cur]
    base = (i - 1) * tile_c
    iota128 = jax.lax.broadcasted_iota(jnp.int32, (1, 128), 1)
    se_r = None
    lt_r = None
    lb_r = None
    for k in range(tile_c // 128):
        s = prev[:, k * 128:(k + 1) * 128]
        m = lab_ref[0:b, :] == (iota128 + (base + k * 128))
        e_k = jnp.exp2(s)
        lt_k = jnp.where(m, s[0:b, :], 0.0)
        lb_k = jnp.where(m, s[b:, :], 0.0)
        se_r = e_k if se_r is None else se_r + e_k
        lt_r = lt_k if lt_r is None else lt_r + lt_k
        lb_r = lb_k if lb_r is None else lb_r + lb_k
    acc_se_ref[...] += se_r
    acc_ll_ref[0:b, :] += lt_r
    acc_ll_ref[b:, :] += lb_r

    @pl.when(i == grid_n)
    def _finalize():
        b = x_ref.shape[0] // 2
        n_extra = grid_n * tile_c - num_classes + tile_c
        se = jnp.sum(acc_se_ref[...], axis=1, keepdims=True) - n_extra
        ll = jnp.sum(acc_ll_ref[...], axis=1, keepdims=True)
        ce = jnp.log(se) - ll * LN2
        v_loss = jnp.sum(ce[:b]) / b
        t_loss = jnp.sum(ce[b:]) / b

        sim = jax.lax.dot_general(
            x_ref[0:b, :], tt_ref[...],
            (((1,), (0,)), ((), ())),
            preferred_element_type=jnp.float32)
        lmat = lab_ref[0:b, :] == labrow_ref[0:1, :]
        loss_pos = jnp.log1p(jnp.exp(-SCALE_POS * (sim - ALPHA)))
        loss_neg = jnp.log1p(jnp.exp(SCALE_NEG * (sim - BETA)))
        ga = 2.0 * jnp.sum(jnp.where(lmat, loss_pos, loss_neg)) / b

        col = jax.lax.broadcasted_iota(jnp.int32, (8, 128), 1)
        res = jnp.where(col == 0, v_loss + t_loss,
              jnp.where(col == 1, ga,
              jnp.where(col == 2, v_loss, t_loss)))
        out_ref[...] = res


def kernel(visual_embed, textual_embed, labels, W):
    batch, feat = visual_embed.shape
    num_classes = W.shape[1]
    tile_c = 1024
    grid_n = (num_classes + tile_c - 1) // tile_c

    x = jnp.concatenate([visual_embed, textual_embed], axis=0)
    x = x.astype(jnp.bfloat16)
    tt = textual_embed.T.astype(jnp.bfloat16)
    lab = jnp.concatenate([labels, labels]).astype(jnp.int32)
    lab = lab.reshape(2 * batch, 1)
    labrow = jnp.broadcast_to(labels.astype(jnp.int32)[None, :], (8, batch))

    out = pl.pallas_call(
        functools.partial(_loss_kernel, num_classes, grid_n, tile_c),
        grid=(grid_n + 1,),
        in_specs=[
            pl.BlockSpec((2 * batch, feat), lambda i: (0, 0)),
            pl.BlockSpec((feat, tile_c),
                         lambda i: (0, jnp.minimum(i, grid_n - 1))),
            pl.BlockSpec((2 * batch, 1), lambda i: (0, 0)),
            pl.BlockSpec((8, batch), lambda i: (0, 0)),
            pl.BlockSpec((feat, batch), lambda i: (0, 0)),
        ],
        out_specs=pl.BlockSpec((8, 128), lambda i: (0, 0)),
        out_shape=jax.ShapeDtypeStruct((8, 128), jnp.float32),
        scratch_shapes=[
            pltpu.VMEM((2 * batch, 128), jnp.float32),
            pltpu.VMEM((2 * batch, 128), jnp.float32),
            pltpu.VMEM((2, 2 * batch, tile_c), jnp.float32),
        ],
    )(x, W, lab, labrow, tt)

    instance_loss = out[0, 0]
    global_align_loss = out[0, 1]
    v_loss = out[0, 2]
    t_loss = out[0, 3]
    return (instance_loss, global_align_loss, v_loss, t_loss)


# speedup vs baseline: 1.2905x; 1.0348x over previous
import functools

import jax
import jax.numpy as jnp
from jax.experimental import pallas as pl
from jax.experimental.pallas import tpu as pltpu

SCALE = 28.0
ALPHA = 0.6
BETA = 0.4
SCALE_POS = 10.0
SCALE_NEG = 40.0
LOG2E = 1.4426950408889634
LN2 = 0.6931471805599453


def _loss_kernel(num_classes, grid_n, tile_c,
                 x_ref, w_ref, lab_ref, labrow_ref, tt_ref,
                 out_ref, acc_se_ref, acc_ll_ref, dots_ref):
    i = pl.program_id(0)
    cur = jax.lax.rem(i, 2)

    @pl.when(i == 0)
    def _init():
        acc_se_ref[...] = jnp.zeros_like(acc_se_ref)
        acc_ll_ref[...] = jnp.zeros_like(acc_ll_ref)
        dots_ref[1] = jnp.zeros_like(dots_ref[1])

    cid = jax.lax.broadcasted_iota(jnp.int32, (1, tile_c), 1) + i * tile_c
    valid = cid < num_classes
    w = jnp.where(valid, w_ref[...], 0.0)
    sumsq = jnp.sum(w * w, axis=0, keepdims=True)
    scale = (SCALE * LOG2E) * jax.lax.rsqrt(jnp.maximum(sumsq, 1e-30))
    ws = (w * scale).astype(jnp.bfloat16)
    dots_ref[cur] = jax.lax.dot_general(
        x_ref[...], ws,
        (((1,), (0,)), ((), ())),
        preferred_element_type=jnp.float32)

    prev = dots_ref[1 - cur]
    base = (i - 1) * tile_c
    iota128 = jax.lax.broadcasted_iota(jnp.int32, (1, 128), 1)
    se_r = None
    ll_r = None
    for k in range(tile_c // 128):
        s = prev[:, k * 128:(k + 1) * 128]
        m = lab_ref[...] == (iota128 + (base + k * 128))
        e_k = jnp.exp2(s)
        l_k = jnp.where(m, s, 0.0)
        se_r = e_k if se_r is None else se_r + e_k
        ll_r = l_k if ll_r is None else ll_r + l_k
    acc_se_ref[...] += se_r
    acc_ll_ref[...] += ll_r

    @pl.when(i == grid_n)
    def _finalize():
        b = x_ref.shape[0] // 2
        n_extra = grid_n * tile_c - num_classes + tile_c
        se = jnp.sum(acc_se_ref[...], axis=1, keepdims=True) - n_extra
        ll = jnp.sum(acc_ll_ref[...], axis=1, keepdims=True)
        ce = jnp.log(se) - ll * LN2
        v_loss = jnp.sum(ce[:b]) / b
        t_loss = jnp.sum(ce[b:]) / b

        sim = jax.lax.dot_general(
            x_ref[0:b, :], tt_ref[...],
            (((1,), (0,)), ((), ())),
            preferred_element_type=jnp.float32)
        lmat = lab_ref[0:b, :] == labrow_ref[0:1, :]
        loss_pos = jnp.log1p(jnp.exp(-SCALE_POS * (sim - ALPHA)))
        loss_neg = jnp.log1p(jnp.exp(SCALE_NEG * (sim - BETA)))
        ga = 2.0 * jnp.sum(jnp.where(lmat, loss_pos, loss_neg)) / b

        col = jax.lax.broadcasted_iota(jnp.int32, (8, 128), 1)
        res = jnp.where(col == 0, v_loss + t_loss,
              jnp.where(col == 1, ga,
              jnp.where(col == 2, v_loss, t_loss)))
        out_ref[...] = res


def kernel(visual_embed, textual_embed, labels, W):
    batch, feat = visual_embed.shape
    num_classes = W.shape[1]
    tile_c = 1024
    grid_n = (num_classes + tile_c - 1) // tile_c

    x = jnp.concatenate([visual_embed, textual_embed], axis=0)
    x = x.astype(jnp.bfloat16)
    tt = textual_embed.T.astype(jnp.bfloat16)
    lab = jnp.concatenate([labels, labels]).astype(jnp.int32)
    lab = lab.reshape(2 * batch, 1)
    labrow = jnp.broadcast_to(labels.astype(jnp.int32)[None, :], (8, batch))

    out = pl.pallas_call(
        functools.partial(_loss_kernel, num_classes, grid_n, tile_c),
        grid=(grid_n + 1,),
        in_specs=[
            pl.BlockSpec((2 * batch, feat), lambda i: (0, 0)),
            pl.BlockSpec((feat, tile_c),
                         lambda i: (0, jnp.minimum(i, grid_n - 1))),
            pl.BlockSpec((2 * batch, 1), lambda i: (0, 0)),
            pl.BlockSpec((8, batch), lambda i: (0, 0)),
            pl.BlockSpec((feat, batch), lambda i: (0, 0)),
        ],
        out_specs=pl.BlockSpec((8, 128), lambda i: (0, 0)),
        out_shape=jax.ShapeDtypeStruct((8, 128), jnp.float32),
        scratch_shapes=[
            pltpu.VMEM((2 * batch, 128), jnp.float32),
            pltpu.VMEM((2 * batch, 128), jnp.float32),
            pltpu.VMEM((2, 2 * batch, tile_c), jnp.float32),
        ],
    )(x, W, lab, labrow, tt)

    instance_loss = out[0, 0]
    global_align_loss = out[0, 1]
    v_loss = out[0, 2]
    t_loss = out[0, 3]
    return (instance_loss, global_align_loss, v_loss, t_loss)
